# baseline (speedup 1.0000x reference)
import functools

import jax
import jax.numpy as jnp
from jax import lax
from jax.experimental import pallas as pl
from jax.experimental.pallas import tpu as pltpu
from jax.experimental.pallas import tpu_sc as plsc

_NC = 2
_NS = 16
_NW = _NC * _NS

_B = 16384
_D = 32
_NROWS = 1000000
_BPW = _B // _NW
_G = 16
_NG = _BPW // _G
_LAST = (_NROWS // 128) * 128
_TAIL = _NROWS - _LAST


_GH = 8
_NGH = _BPW // _GH


def _gather_body(uid_hbm, iid_hbm, ut_hbm, it_hbm, out_u, out_i,
                 uid_v, iid_v, slabs, obuf, sem_a, sem_b):
    wid = lax.axis_index("s") * _NC + lax.axis_index("c")
    base = wid * _BPW
    pltpu.sync_copy(uid_hbm.at[pl.ds(base, _BPW)], uid_v.at[pl.ds(0, _BPW)])
    pltpu.sync_copy(iid_hbm.at[pl.ds(base, _BPW)], iid_v.at[pl.ds(0, _BPW)])

    lane = lax.iota(jnp.int32, 16)
    mask8 = lane < _GH
    slot8 = (lane & 7) * 128

    def one_table(tab_hbm, ids_v, out_hbm):
        def fire(g, half, sem):
            k0 = g * _GH
            gcol = ids_v[pl.ds(k0, 16)] & ~127
            for j in range(_GH):
                col0 = pl.multiple_of(gcol[j], 128)
                pltpu.async_copy(
                    tab_hbm.at[:, pl.ds(col0, 128)],
                    slabs.at[:, pl.ds((half + j) * 128, 128)], sem)

        def drain(half, sem):
            pltpu.make_async_copy(
                tab_hbm.at[:, pl.ds(0, _GH * 128)],
                slabs.at[:, pl.ds(half * 128, _GH * 128)], sem).wait()

        def extract(g, half):
            k0 = g * _GH
            gvec = ids_v[pl.ds(k0, 16)]
            colidx = half * 128 + slot8 + (gvec & 127)
            colv = k0 + lane
            for f in range(_D):
                rowv = jnp.full((16,), f, jnp.int32)
                vals = plsc.load_gather(slabs, [rowv, colidx])
                plsc.store_scatter(obuf, [rowv, colv], vals, mask=mask8)

        fire(0, 0, sem_a)
        fire(1, _GH, sem_b)

        def body(i, c):
            g = 2 * i
            drain(0, sem_a)
            extract(g, 0)

            @pl.when(g + 2 < _NGH)
            def _():
                fire(g + 2, 0, sem_a)

            drain(_GH, sem_b)
            extract(g + 1, _GH)

            @pl.when(g + 3 < _NGH)
            def _():
                fire(g + 3, _GH, sem_b)
            return c

        lax.fori_loop(0, _NGH // 2, body, 0)
        pltpu.sync_copy(obuf, out_hbm.at[:, pl.ds(base, _BPW)])

    one_table(ut_hbm, uid_v, out_u)
    one_table(it_hbm, iid_v, out_i)


@functools.cache
def _make_gather():
    return pl.kernel(
        _gather_body,
        out_type=(
            jax.ShapeDtypeStruct((_D, _B), jnp.float32),
            jax.ShapeDtypeStruct((_D, _B), jnp.float32),
        ),
        mesh=plsc.VectorSubcoreMesh(core_axis_name="c", subcore_axis_name="s"),
        scratch_types=[
            pltpu.VMEM((_BPW + 16,), jnp.int32),
            pltpu.VMEM((_BPW + 16,), jnp.int32),
            pltpu.VMEM((_D, _G * 128), jnp.float32),
            pltpu.VMEM((_D, _BPW), jnp.float32),
            pltpu.SemaphoreType.DMA,
            pltpu.SemaphoreType.DMA,
        ],
        compiler_params=pltpu.CompilerParams(
            use_tc_tiling_on_sc=True, needs_layout_passes=False),
    )


def _mlp_body(ut_ref, vt_ref, w1u_ref, w1i_ref, b1_ref, w2_ref, b2_ref,
              w3_ref, b3_ref, out_ref):
    h = w1u_ref[:] @ ut_ref[:] + w1i_ref[:] @ vt_ref[:] + b1_ref[:]
    h = jnp.maximum(h, 0.0)
    h2 = jnp.maximum(w2_ref[:] @ h + b2_ref[:], 0.0)
    o = w3_ref[:] @ h2
    out_ref[:] = o[0] + b3_ref[0]


def _mlp(ut, vt, w1u, w1i, b1, w2, b2, w3, b3, block_b=2048):
    nb = _B // block_b
    return pl.pallas_call(
        _mlp_body,
        grid=(nb,),
        in_specs=[
            pl.BlockSpec((_D, block_b), lambda i: (0, i)),
            pl.BlockSpec((_D, block_b), lambda i: (0, i)),
            pl.BlockSpec(w1u.shape, lambda i: (0, 0)),
            pl.BlockSpec(w1i.shape, lambda i: (0, 0)),
            pl.BlockSpec(b1.shape, lambda i: (0, 0)),
            pl.BlockSpec(w2.shape, lambda i: (0, 0)),
            pl.BlockSpec(b2.shape, lambda i: (0, 0)),
            pl.BlockSpec(w3.shape, lambda i: (0, 0)),
            pl.BlockSpec(b3.shape, lambda i: (0,)),
        ],
        out_specs=pl.BlockSpec((block_b,), lambda i: (i,)),
        out_shape=jax.ShapeDtypeStruct((_B,), jnp.float32),
    )(ut, vt, w1u, w1i, b1, w2, b2, w3, b3)


def kernel(user_ids, item_ids, user_table, item_table, W1, b1, W2, b2, W3, b3):
    ut_t = user_table.T
    it_t = item_table.T
    u_t, v_t = _make_gather()(user_ids, item_ids, ut_t, it_t)
    w1u = W1[:, :_D]
    w1i = W1[:, _D:]
    return _mlp(u_t, v_t, w1u, w1i, b1[:, None], W2, b2[:, None], W3, b3)
